# Initial kernel scaffold; baseline (speedup 1.0000x reference)
import jax
import jax.numpy as jnp
from jax.experimental import pallas as pl


def kernel(x, X_train, y_train):
    raise NotImplementedError("write your pallas kernel here")


# speedup vs baseline: 3.3216x; 3.3216x over previous
import functools

import jax
import jax.numpy as jnp
from jax import lax
from jax.experimental import pallas as pl
from jax.experimental.pallas import tpu as pltpu
from jax.experimental.pallas import tpu_sc as plsc

N = 100000
D = 128
Q = 1024
LANES = 128
CHUNKS_PER_BLOCK = 16
NB = CHUNKS_PER_BLOCK * LANES
NUM_BLOCKS = (N + NB - 1) // NB
TOTAL_CHUNKS = NUM_BLOCKS * CHUNKS_PER_BLOCK
KTOP = 5
NCAND = KTOP * LANES


def _phase1_body(x_ref, qsq_ref, k_ref, ksq_ref, out_v_ref, out_i_ref, sv, si):
    nb = pl.program_id(0)

    @pl.when(nb == 0)
    def _init():
        sv[...] = jnp.full(sv.shape, jnp.inf, dtype=jnp.float32)
        si[...] = jnp.zeros(si.shape, dtype=jnp.int32)

    mm = lax.dot_general(
        x_ref[...], k_ref[...], (((1,), (1,)), ((), ())),
        preferred_element_type=jnp.float32)
    t = qsq_ref[...] - 2.0 * mm

    for s in range(CHUNKS_PER_BLOCK):
        sl = slice(s * LANES, (s + 1) * LANES)
        c = nb * CHUNKS_PER_BLOCK + s
        d = t[:, sl] + ksq_ref[s:s + 1, :]

        v = [sv[r] for r in range(KTOP)]
        i = [si[r] for r in range(KTOP)]
        b = [d < v[r] for r in range(KTOP)]
        for r in range(KTOP - 1, 0, -1):
            sv[r] = jnp.where(b[r - 1], v[r - 1], jnp.where(b[r], d, v[r]))
            si[r] = jnp.where(b[r - 1], i[r - 1], jnp.where(b[r], c, i[r]))
        sv[0] = jnp.where(b[0], d, v[0])
        si[0] = jnp.where(b[0], c, i[0])

    @pl.when(nb == NUM_BLOCKS - 1)
    def _emit():
        gi = lax.broadcasted_iota(jnp.int32, (Q, LANES), 1)
        for r in range(KTOP):
            out_v_ref[:, r * LANES:(r + 1) * LANES] = sv[r]
            out_i_ref[:, r * LANES:(r + 1) * LANES] = si[r] * LANES + gi


@jax.jit
def _phase1(x, X_train):
    qsq = jnp.sum(x * x, axis=1, keepdims=True)
    ksq = jnp.sum(X_train * X_train, axis=1)
    ksq = jnp.concatenate(
        [ksq, jnp.full((TOTAL_CHUNKS * LANES - N,), jnp.inf, jnp.float32)])
    ksq = ksq.reshape(TOTAL_CHUNKS, LANES)
    return pl.pallas_call(
        _phase1_body,
        grid=(NUM_BLOCKS,),
        in_specs=[
            pl.BlockSpec((Q, D), lambda nb: (0, 0)),
            pl.BlockSpec((Q, 1), lambda nb: (0, 0)),
            pl.BlockSpec((NB, D), lambda nb: (nb, 0)),
            pl.BlockSpec((CHUNKS_PER_BLOCK, LANES), lambda nb: (nb, 0)),
        ],
        out_specs=[
            pl.BlockSpec((Q, NCAND), lambda nb: (0, 0)),
            pl.BlockSpec((Q, NCAND), lambda nb: (0, 0)),
        ],
        out_shape=[
            jax.ShapeDtypeStruct((Q, NCAND), jnp.float32),
            jax.ShapeDtypeStruct((Q, NCAND), jnp.int32),
        ],
        scratch_shapes=[
            pltpu.VMEM((KTOP, Q, LANES), jnp.float32),
            pltpu.VMEM((KTOP, Q, LANES), jnp.int32),
        ],
    )(x, qsq, X_train, ksq)


def _phase2_stub(cand_v, cand_i, y_train):
    idx_sorted = jnp.argsort(cand_i, axis=-1)
    v_by_i = jnp.take_along_axis(cand_v, idx_sorted, axis=-1)
    i_by_i = jnp.take_along_axis(cand_i, idx_sorted, axis=-1)
    val_order = jnp.argsort(v_by_i, axis=-1, stable=True)[:, :KTOP]
    top_i = jnp.take_along_axis(i_by_i, val_order, axis=-1)
    return jnp.mean(jnp.take(y_train, top_i, axis=0), axis=1)


NW = 32
QW = Q // NW
NVR = NCAND // 16
YROWS = N // 16
_BIG = 2**31 - 1


def _xlane(v, perm):
    return lax.gather(
        v, perm[:, None],
        lax.GatherDimensionNumbers(offset_dims=(), collapsed_slice_dims=(0,),
                                   start_index_map=(0,)),
        slice_sizes=(1,), mode=lax.GatherScatterMode.PROMISE_IN_BOUNDS)


def _bfly(v, op):
    iota = lax.iota(jnp.int32, 16)
    for sh in (8, 4, 2, 1):
        v = op(v, _xlane(v, iota ^ sh))
    return v


def _phase2_body(cv_hbm, ci_hbm, y_hbm, out_hbm,
                 y_v, val_v, idx_v, pred_v, sem):
    del sem
    wid = lax.axis_index("s") * 2 + lax.axis_index("c")
    pltpu.sync_copy(y_hbm, y_v)

    lane_iota = lax.iota(jnp.int32, 16)

    for h in range(QW // 16):
        base = wid * QW + h * 16
        pltpu.sync_copy(cv_hbm.at[pl.ds(base, 16)], val_v)
        pltpu.sync_copy(ci_hbm.at[pl.ds(base, 16)], idx_v)

        def per_query(q, pacc):
            jvec = jnp.zeros((16,), jnp.int32)
            for e in range(KTOP):
                def scan(j, carry):
                    mv, mi = carry
                    v = val_v[q, pl.ds(j * 16, 16)]
                    ii = idx_v[q, pl.ds(j * 16, 16)]
                    take = (v < mv) | ((v == mv) & (ii < mi))
                    return (jnp.where(take, v, mv), jnp.where(take, ii, mi))

                mv0 = jnp.full((16,), jnp.inf, jnp.float32)
                mi0 = jnp.full((16,), _BIG, jnp.int32)
                mv, mi = lax.fori_loop(0, NVR, scan, (mv0, mi0))
                m = _bfly(mv, jnp.minimum)
                j_sel = _bfly(jnp.where(mv == m, mi, _BIG), jnp.minimum)

                def knock(j, _):
                    ii = idx_v[q, pl.ds(j * 16, 16)]
                    v = val_v[q, pl.ds(j * 16, 16)]
                    val_v[q, pl.ds(j * 16, 16)] = jnp.where(
                        ii == j_sel, jnp.float32(jnp.inf), v)
                    return 0

                lax.fori_loop(0, NVR, knock, 0)
                jvec = jnp.where(lane_iota == e, j_sel, jvec)

            yv = plsc.load_gather(y_v, [jvec])
            yv = jnp.where(lane_iota < KTOP, yv, jnp.float32(0.0))
            tot = _bfly(yv, jnp.add) * jnp.float32(0.2)
            return jnp.where(lane_iota == q, tot, pacc)

        pacc = lax.fori_loop(0, 16, per_query, jnp.zeros((16,), jnp.float32))
        pred_v[pl.ds(0, 16)] = pacc
        pltpu.sync_copy(pred_v, out_hbm.at[pl.ds(base, 16)])


@jax.jit
def _phase2_sc(cand_v, cand_i, y1d):
    mesh = plsc.VectorSubcoreMesh(core_axis_name="c", subcore_axis_name="s")
    run = functools.partial(
        pl.kernel,
        mesh=mesh,
        out_type=jax.ShapeDtypeStruct((Q,), jnp.float32),
        scratch_types=[
            pltpu.VMEM((N,), jnp.float32),
            pltpu.VMEM((16, NCAND), jnp.float32),
            pltpu.VMEM((16, NCAND), jnp.int32),
            pltpu.VMEM((16,), jnp.float32),
            pltpu.SemaphoreType.DMA,
        ],
        compiler_params=pltpu.CompilerParams(needs_layout_passes=False),
    )(_phase2_body)
    return run(cand_v, cand_i, y1d)


def kernel(x, X_train, y_train):
    cand_v, cand_i = _phase1(x, X_train)
    return _phase2_sc(cand_v, cand_i, y_train)
